# baseline (speedup 1.0000x reference)
import functools

import jax
import jax.numpy as jnp
from jax import lax
from jax.experimental import pallas as pl
from jax.experimental.pallas import tpu as pltpu
from jax.experimental.pallas import tpu_sc as plsc

_B = 16384
_D = 8
_NW = 32
_BPW = _B // _NW
_CH = 128
_NCH = _BPW // _CH

_EPS = 1e-5


_NU = 1000000
_NM = 100000


def _sc_gather(users, movies, planes_cat):
    mesh = plsc.VectorSubcoreMesh(core_axis_name="c", subcore_axis_name="s")

    @functools.partial(
        pl.kernel,
        mesh=mesh,
        out_type=jax.ShapeDtypeStruct((2 * _D, _B), jnp.float32),
        scratch_types=[
            pltpu.VMEM((_BPW,), jnp.int32),
            pltpu.VMEM((_BPW,), jnp.int32),
            pltpu.VMEM((_D * _NCH, _CH), jnp.int32),
            pltpu.VMEM((_D * _NCH, _CH), jnp.int32),
            pltpu.VMEM((2 * _D, _BPW), jnp.float32),
            pltpu.SemaphoreType.DMA,
        ],
    )
    def k(users_hbm, movies_hbm, pc_hbm, out_hbm,
          idx_u, idx_m, idxb_u, idxb_m, xt, sem):
        wid = lax.axis_index("s") * 2 + lax.axis_index("c")
        base = wid * _BPW
        pltpu.sync_copy(users_hbm.at[pl.ds(base, _BPW)], idx_u)
        pltpu.sync_copy(movies_hbm.at[pl.ds(base, _BPW)], idx_m)
        for c in range(_NCH):
            for g in range(_CH // 16):
                off = c * _CH + g * 16
                vu = idx_u[pl.ds(off, 16)]
                vm = idx_m[pl.ds(off, 16)]
                for f in range(_D):
                    idxb_u[f * _NCH + c, pl.ds(g * 16, 16)] = vu + f * _NU
                    idxb_m[f * _NCH + c, pl.ds(g * 16, 16)] = (
                        vm + (_D * _NU + f * _NM))
        copies = []
        for f in range(_D):
            for c in range(_NCH):
                sl = pl.ds(c * _CH, _CH)
                copies.append(pltpu.async_copy(
                    pc_hbm.at[idxb_u.at[f * _NCH + c]], xt.at[f, sl], sem))
                copies.append(pltpu.async_copy(
                    pc_hbm.at[idxb_m.at[f * _NCH + c]], xt.at[_D + f, sl], sem))
        for cp in copies:
            cp.wait()
        pltpu.sync_copy(xt, out_hbm.at[:, pl.ds(base, _BPW)])

    return k(users, movies, planes_cat)


def _mlp_body(x_ref, w0_ref, b0_ref, g0_ref, be0_ref,
              w1_ref, b1_ref, g1_ref, be1_ref,
              w2_ref, b2_ref, g2_ref, be2_ref,
              w3_ref, b3_ref, o_ref):
    inv = 1.0 / jnp.sqrt(1.0 + _EPS)

    def layer(h, w_ref, b_ref, g_ref, be_ref):
        z = jnp.dot(w_ref[...], h, preferred_element_type=jnp.float32)
        z = z + b_ref[...]
        z = jnp.maximum(z, 0.0)
        return (z * inv) * g_ref[...] + be_ref[...]

    x = x_ref[...]
    h = layer(x, w0_ref, b0_ref, g0_ref, be0_ref)
    h = layer(h, w1_ref, b1_ref, g1_ref, be1_ref)
    h = layer(h, w2_ref, b2_ref, g2_ref, be2_ref)
    y = jnp.dot(w3_ref[...], h, preferred_element_type=jnp.float32)
    o_ref[...] = y + b3_ref[...]


def _tc_mlp(x, W0, b0, g0, be0, W1, b1, g1, be1, W2, b2, g2, be2, W3, b3):
    col = lambda v: v.reshape(-1, 1)
    args = (x, W0, col(b0), col(g0), col(be0),
            W1, col(b1), col(g1), col(be1),
            W2, col(b2), col(g2), col(be2),
            W3, col(b3))
    return pl.pallas_call(
        _mlp_body,
        out_shape=jax.ShapeDtypeStruct((1, _B), jnp.float32),
    )(*args)


def kernel(users, movies, user_table, movie_table,
           W0, b0, g0, be0, W1, b1, g1, be1,
           W2, b2, g2, be2, W3, b3):
    planes_cat = jnp.concatenate(
        [user_table[:, f] for f in range(_D)]
        + [movie_table[:, f] for f in range(_D)])
    x = _sc_gather(users.astype(jnp.int32), movies.astype(jnp.int32),
                   planes_cat)
    y = _tc_mlp(x, W0, b0, g0, be0, W1, b1, g1, be1, W2, b2, g2, be2, W3, b3)
    return y.reshape(_B, 1)


# speedup vs baseline: 8.6194x; 8.6194x over previous
import functools

import jax
import jax.numpy as jnp
from jax import lax
from jax.experimental import pallas as pl
from jax.experimental.pallas import tpu as pltpu
from jax.experimental.pallas import tpu_sc as plsc

_B = 16384
_D = 8
_NW = 32
_BPW = _B // _NW
_CH = 128
_NCH = _BPW // _CH

_EPS = 1e-5


def _tile_view(table):
    n = table.shape[0]
    npad = (-n) % 128
    tp = jnp.pad(table, ((0, npad), (0, 0)))
    return tp.reshape(-1, 128, _D).transpose(0, 2, 1).reshape(-1)


def _sc_gather(users, movies, u_v, m_v):
    mesh = plsc.VectorSubcoreMesh(core_axis_name="c", subcore_axis_name="s")

    @functools.partial(
        pl.kernel,
        mesh=mesh,
        out_type=jax.ShapeDtypeStruct((2 * _D, _B), jnp.float32),
        scratch_types=[
            pltpu.VMEM((_BPW,), jnp.int32),
            pltpu.VMEM((_BPW,), jnp.int32),
            pltpu.VMEM((_D * _NCH, _CH), jnp.int32),
            pltpu.VMEM((_D * _NCH, _CH), jnp.int32),
            pltpu.VMEM((2 * _D, _BPW), jnp.float32),
            pltpu.SemaphoreType.DMA,
        ],
    )
    def k(users_hbm, movies_hbm, uv_hbm, mv_hbm, out_hbm,
          idx_u, idx_m, idxb_u, idxb_m, xt, sem):
        wid = lax.axis_index("s") * 2 + lax.axis_index("c")
        base = wid * _BPW
        pltpu.sync_copy(users_hbm.at[pl.ds(base, _BPW)], idx_u)
        pltpu.sync_copy(movies_hbm.at[pl.ds(base, _BPW)], idx_m)
        for c in range(_NCH):
            for g in range(_CH // 16):
                off = c * _CH + g * 16
                vu = idx_u[pl.ds(off, 16)]
                vm = idx_m[pl.ds(off, 16)]
                bu = (lax.shift_left(lax.shift_right_logical(vu, 7), 10)
                      + lax.bitwise_and(vu, 127))
                bm = (lax.shift_left(lax.shift_right_logical(vm, 7), 10)
                      + lax.bitwise_and(vm, 127))
                for f in range(_D):
                    idxb_u[f * _NCH + c, pl.ds(g * 16, 16)] = bu + f * 128
                    idxb_m[f * _NCH + c, pl.ds(g * 16, 16)] = bm + f * 128
        copies = []
        for f in range(_D):
            for c in range(_NCH):
                sl = pl.ds(c * _CH, _CH)
                copies.append(pltpu.async_copy(
                    uv_hbm.at[idxb_u.at[f * _NCH + c]], xt.at[f, sl], sem))
                copies.append(pltpu.async_copy(
                    mv_hbm.at[idxb_m.at[f * _NCH + c]], xt.at[_D + f, sl], sem))
        for cp in copies:
            cp.wait()
        pltpu.sync_copy(xt, out_hbm.at[:, pl.ds(base, _BPW)])

    return k(users, movies, u_v, m_v)


def _mlp_body(x_ref, w0_ref, b0_ref, g0_ref, be0_ref,
              w1_ref, b1_ref, g1_ref, be1_ref,
              w2_ref, b2_ref, g2_ref, be2_ref,
              w3_ref, b3_ref, o_ref):
    inv = 1.0 / jnp.sqrt(1.0 + _EPS)

    def layer(h, w_ref, b_ref, g_ref, be_ref):
        z = jnp.dot(w_ref[...], h, preferred_element_type=jnp.float32)
        z = z + b_ref[...]
        z = jnp.maximum(z, 0.0)
        return (z * inv) * g_ref[...] + be_ref[...]

    x = x_ref[...]
    h = layer(x, w0_ref, b0_ref, g0_ref, be0_ref)
    h = layer(h, w1_ref, b1_ref, g1_ref, be1_ref)
    h = layer(h, w2_ref, b2_ref, g2_ref, be2_ref)
    y = jnp.dot(w3_ref[...], h, preferred_element_type=jnp.float32)
    o_ref[...] = y + b3_ref[...]


def _tc_mlp(x, W0, b0, g0, be0, W1, b1, g1, be1, W2, b2, g2, be2, W3, b3):
    col = lambda v: v.reshape(-1, 1)
    args = (x, W0, col(b0), col(g0), col(be0),
            W1, col(b1), col(g1), col(be1),
            W2, col(b2), col(g2), col(be2),
            W3, col(b3))
    return pl.pallas_call(
        _mlp_body,
        out_shape=jax.ShapeDtypeStruct((1, _B), jnp.float32),
    )(*args)


def kernel(users, movies, user_table, movie_table,
           W0, b0, g0, be0, W1, b1, g1, be1,
           W2, b2, g2, be2, W3, b3):
    x = _sc_gather(users.astype(jnp.int32), movies.astype(jnp.int32),
                   _tile_view(user_table), _tile_view(movie_table))
    y = _tc_mlp(x, W0, b0, g0, be0, W1, b1, g1, be1, W2, b2, g2, be2, W3, b3)
    return y.reshape(_B, 1)
